# baseline (speedup 1.0000x reference)
import functools

import jax
import jax.numpy as jnp
from jax import lax
from jax.experimental import pallas as pl
from jax.experimental.pallas import tpu as pltpu
from jax.experimental.pallas import tpu_sc as plsc

N = 10000
E = 320000
VOCAB = 1000
F = 128
HID = 128
LAT = 64

NC = 2
NS = 16
NW = NC * NS
EPT = E // NW
RPT = N // NS

_MESH = dict(core_axis_name="c", subcore_axis_name="s")


def _deg_partials(dst):
    @functools.partial(
        pl.kernel,
        out_type=jax.ShapeDtypeStruct((NW, N), jnp.float32),
        mesh=plsc.VectorSubcoreMesh(**_MESH),
        scratch_types=[
            pltpu.VMEM((N,), jnp.float32),
            pltpu.VMEM((2000,), jnp.int32),
        ],
        compiler_params=pltpu.CompilerParams(needs_layout_passes=False),
    )
    def k(dst_hbm, out_hbm, hist, dbuf):
        wid = lax.axis_index("c") * NS + lax.axis_index("s")
        zeros16 = jnp.zeros((16,), jnp.float32)
        ones16 = jnp.full((16,), 1.0, jnp.float32)

        def zbody(i, c):
            hist[pl.ds(i * 16, 16)] = zeros16
            return c

        lax.fori_loop(0, N // 16, zbody, 0)

        ebase = wid * EPT

        def cbody(cc, c):
            b = pl.multiple_of(ebase + cc * 2000, 8)
            pltpu.sync_copy(dst_hbm.at[pl.ds(b, 2000)], dbuf)

            def ibody(j, c2):
                idx = dbuf[pl.ds(j * 16, 16)]
                plsc.addupdate_scatter(hist, [idx], ones16)
                return c2

            lax.fori_loop(0, 2000 // 16, ibody, 0)
            return c

        lax.fori_loop(0, EPT // 2000, cbody, 0)
        pltpu.sync_copy(hist, out_hbm.at[wid])

    return k(dst)


def _emb_gather(x, table):
    @functools.partial(
        pl.kernel,
        out_type=jax.ShapeDtypeStruct((N, F), jnp.float32),
        mesh=plsc.VectorSubcoreMesh(**_MESH),
        scratch_types=[
            pltpu.VMEM((312,), jnp.int32),
            pltpu.VMEM((104, F), jnp.float32),
            pltpu.VMEM((16,), jnp.int32),
            pltpu.VMEM((16, F), jnp.float32),
            pltpu.SemaphoreType.DMA,
        ],
    )
    def k(x_hbm, tab_hbm, out_hbm, xbuf, rows, xbuf2, rows2, sem):
        wid = lax.axis_index("c") * NS + lax.axis_index("s")
        base = pl.multiple_of(wid * 312, 8)
        pltpu.sync_copy(x_hbm.at[pl.ds(base, 312)], xbuf)
        for kk in range(3):
            pltpu.async_copy(
                tab_hbm.at[xbuf.at[pl.ds(kk * 104, 104)]], rows, sem
            ).wait()
            pltpu.sync_copy(rows, out_hbm.at[pl.ds(base + kk * 104, 104)])

        @pl.when(wid == 0)
        def _tail():
            pltpu.sync_copy(x_hbm.at[pl.ds(312 * NW, 16)], xbuf2)
            pltpu.async_copy(tab_hbm.at[xbuf2], rows2, sem).wait()
            pltpu.sync_copy(rows2, out_hbm.at[pl.ds(312 * NW, 16)])

    return k(x, table)


NPAD = N + 16
CPT = 80
NGRP = CPT // 8


def _edge_scatter(src2d, dst2d, hwp):
    @functools.partial(
        pl.kernel,
        out_type=jax.ShapeDtypeStruct((NC, N, F), jnp.float32),
        mesh=plsc.VectorSubcoreMesh(**_MESH),
        scratch_types=[
            [pltpu.VMEM((128,), jnp.int32)] * 2,
            [pltpu.VMEM((128,), jnp.int32)] * 2,
            pltpu.VMEM((128, F), jnp.float32),
            pltpu.VMEM((128, F), jnp.float32),
            pltpu.VMEM_SHARED((NPAD, F), jnp.float32),
            [pltpu.SemaphoreType.DMA] * 2,
        ],
    )
    def k(src_hbm, dst_hbm, hwp_hbm, out_hbm,
          sbufs, dbufs, r0, r1, acc, gsems):
        cid = lax.axis_index("c")
        sid = lax.axis_index("s")
        wid = cid * NS + sid
        rows = [r0, r1]
        zeros16 = jnp.zeros((16,), jnp.float32)

        def zbody(i, c):
            for c8 in range(F // 16):
                r0[i, pl.ds(c8 * 16, 16)] = zeros16
            return c

        lax.fori_loop(0, 128, zbody, 0)

        chunks_a = [(0, 128), (128, 128), (256, 128), (384, 128), (512, 112)]
        chunks_b = [(0, 128), (128, 128), (256, 128), (384, 128), (512, 128)]

        @pl.when(sid != NS - 1)
        def _init_a():
            base = pl.multiple_of(sid * 624, 8)
            for off, sz in chunks_a:
                pltpu.sync_copy(r0.at[pl.ds(0, sz)],
                                acc.at[pl.ds(base + off, sz)])

        @pl.when(sid == NS - 1)
        def _init_b():
            for off, sz in chunks_b:
                pltpu.sync_copy(r0.at[pl.ds(0, sz)],
                                acc.at[pl.ds(9360 + off, sz)])

        @pl.when(sid == 0)
        def _init_trash():
            pltpu.sync_copy(r0.at[pl.ds(0, NPAD - N)], acc.at[pl.ds(N, NPAD - N)])

        plsc.subcore_barrier()

        gbase = wid * (CPT * 128)

        def load_idx(chunk, s):
            b = pl.multiple_of(gbase + chunk * 128, 8)
            pltpu.sync_copy(src_hbm.at[pl.ds(b, 128)], sbufs[s])
            pltpu.sync_copy(dst_hbm.at[pl.ds(b, 128)], dbufs[s])

        def pbody(p, c):
            load_idx(p, 0)
            pltpu.async_copy(hwp_hbm.at[sbufs[0]], rows[0], gsems[0]).wait()
            pltpu.sync_copy(rows[0], acc.at[dbufs[0]], add=True)
            return c

        lax.fori_loop(0, CPT, pbody, 0)

        plsc.subcore_barrier()

        @pl.when(sid != NS - 1)
        def _wb_a():
            base = pl.multiple_of(sid * 624, 8)
            for off, sz in chunks_a:
                pltpu.sync_copy(acc.at[pl.ds(base + off, sz)],
                                out_hbm.at[cid, pl.ds(base + off, sz)])

        @pl.when(sid == NS - 1)
        def _wb_b():
            for off, sz in chunks_b:
                pltpu.sync_copy(acc.at[pl.ds(9360 + off, sz)],
                                out_hbm.at[cid, pl.ds(9360 + off, sz)])

    return k(src2d, dst2d, hwp)


def _prep(hist, emb, W1):
    def body(hist_ref, emb_ref, w1_ref, dinv_ref, t1_ref):
        ones = jnp.ones((NW, 1), jnp.float32)
        deg = lax.dot_general(
            hist_ref[...], ones, (((0,), (0,)), ((), ())),
            preferred_element_type=jnp.float32,
            precision=lax.Precision.HIGHEST,
        )
        dinv_ref[...] = lax.rsqrt(deg + 1.0)
        t1_ref[...] = jnp.dot(emb_ref[...], w1_ref[...],
                              preferred_element_type=jnp.float32)

    return pl.pallas_call(
        body,
        out_shape=(
            jax.ShapeDtypeStruct((N, 1), jnp.float32),
            jax.ShapeDtypeStruct((VOCAB, F), jnp.float32),
        ),
    )(hist, emb, W1)


_BLK = 1000
_NB = N // _BLK


def _scale(rows, dinv):
    def body(rows_ref, dinv_ref, out_ref):
        out_ref[...] = rows_ref[...] * dinv_ref[...]

    return pl.pallas_call(
        body,
        grid=(_NB,),
        in_specs=[
            pl.BlockSpec((_BLK, F), lambda i: (i, 0)),
            pl.BlockSpec((_BLK, 1), lambda i: (i, 0)),
        ],
        out_specs=pl.BlockSpec((_BLK, F), lambda i: (i, 0)),
        out_shape=jax.ShapeDtypeStruct((N, F), jnp.float32),
    )(rows, dinv)


def _combine(acc, hwp, dinv, b, W):
    def body(acc_ref, hwp_ref, dinv_ref, b_ref, w_ref, out_ref):
        s = acc_ref[0] + acc_ref[1] + hwp_ref[...]
        h = jnp.maximum(s * dinv_ref[...] + b_ref[...], 0.0)
        out_ref[...] = jnp.dot(h, w_ref[...],
                               preferred_element_type=jnp.float32) * dinv_ref[...]

    return pl.pallas_call(
        body,
        grid=(_NB,),
        in_specs=[
            pl.BlockSpec((NC, _BLK, F), lambda i: (0, i, 0)),
            pl.BlockSpec((_BLK, F), lambda i: (i, 0)),
            pl.BlockSpec((_BLK, 1), lambda i: (i, 0)),
            pl.BlockSpec((1, F), lambda i: (0, 0)),
            pl.BlockSpec((F, F), lambda i: (0, 0)),
        ],
        out_specs=pl.BlockSpec((_BLK, F), lambda i: (i, 0)),
        out_shape=jax.ShapeDtypeStruct((N, F), jnp.float32),
    )(acc, hwp, dinv, b, W)


def _head(acc, hwp, dinv, b2, Wmu, bmu, Wlv, blv, W3, b3, W4, b4, Wc, bc, eps):
    def body(acc_ref, hwp_ref, dinv_ref, b2_ref, wmu_ref, bmu_ref, wlv_ref,
             blv_ref, w3_ref, b3_ref, w4_ref, b4_ref, wc_ref, bc_ref, eps_ref,
             recon_ref, cap_ref, mu_ref, lv_ref):
        s = acc_ref[0] + acc_ref[1] + hwp_ref[...]
        h2 = jnp.maximum(s * dinv_ref[...] + b2_ref[...], 0.0)
        mu = jnp.dot(h2, wmu_ref[...], preferred_element_type=jnp.float32) + bmu_ref[...]
        lv = jnp.dot(h2, wlv_ref[...], preferred_element_type=jnp.float32) + blv_ref[...]
        mu_ref[...] = mu
        lv_ref[...] = lv
        z = mu + eps_ref[...] * jnp.exp(0.5 * lv)
        d = jnp.maximum(jnp.dot(z, w3_ref[...],
                                preferred_element_type=jnp.float32) + b3_ref[...], 0.0)
        recon_ref[...] = jnp.dot(d, w4_ref[...],
                                 preferred_element_type=jnp.float32) + b4_ref[...]
        logit = jnp.dot(z, wc_ref[...],
                        preferred_element_type=jnp.float32) + bc_ref[...]
        cap_ref[...] = jax.nn.sigmoid(logit)

    return pl.pallas_call(
        body,
        grid=(_NB,),
        in_specs=[
            pl.BlockSpec((NC, _BLK, F), lambda i: (0, i, 0)),
            pl.BlockSpec((_BLK, F), lambda i: (i, 0)),
            pl.BlockSpec((_BLK, 1), lambda i: (i, 0)),
            pl.BlockSpec((1, HID), lambda i: (0, 0)),
            pl.BlockSpec((HID, LAT), lambda i: (0, 0)),
            pl.BlockSpec((1, LAT), lambda i: (0, 0)),
            pl.BlockSpec((HID, LAT), lambda i: (0, 0)),
            pl.BlockSpec((1, LAT), lambda i: (0, 0)),
            pl.BlockSpec((LAT, HID), lambda i: (0, 0)),
            pl.BlockSpec((1, HID), lambda i: (0, 0)),
            pl.BlockSpec((HID, VOCAB), lambda i: (0, 0)),
            pl.BlockSpec((1, VOCAB), lambda i: (0, 0)),
            pl.BlockSpec((LAT, 1), lambda i: (0, 0)),
            pl.BlockSpec((1, 1), lambda i: (0, 0)),
            pl.BlockSpec((_BLK, LAT), lambda i: (i, 0)),
        ],
        out_specs=(
            pl.BlockSpec((_BLK, VOCAB), lambda i: (i, 0)),
            pl.BlockSpec((_BLK, 1), lambda i: (i, 0)),
            pl.BlockSpec((_BLK, LAT), lambda i: (i, 0)),
            pl.BlockSpec((_BLK, LAT), lambda i: (i, 0)),
        ),
        out_shape=(
            jax.ShapeDtypeStruct((N, VOCAB), jnp.float32),
            jax.ShapeDtypeStruct((N, 1), jnp.float32),
            jax.ShapeDtypeStruct((N, LAT), jnp.float32),
            jax.ShapeDtypeStruct((N, LAT), jnp.float32),
        ),
    )(acc, hwp, dinv, b2, Wmu, bmu, Wlv, blv, W3, b3, W4, b4, Wc, bc, eps)


def kernel(x, edge_index, emb, W1, b1, W2, b2, Wmu, bmu, Wlv, blv,
           W3, b3, W4, b4, Wc, bc):
    x = x.astype(jnp.int32)
    src = edge_index[0].astype(jnp.int32)
    dst = edge_index[1].astype(jnp.int32)
    npad = CPT * 128 - EPT
    src2d = jnp.pad(src.reshape(NW, EPT), ((0, 0), (0, npad))
                    ).reshape(NW * CPT * 128)
    dst2d = jnp.pad(dst.reshape(NW, EPT), ((0, 0), (0, npad)),
                    constant_values=N).reshape(NW * CPT * 128)

    hist = _deg_partials(dst)
    dinv, T1 = _prep(hist, emb, W1)
    rows = _emb_gather(x, T1)
    hwp1 = _scale(rows, dinv)
    acc1 = _edge_scatter(src2d, dst2d, hwp1)
    hwp2 = _combine(acc1, hwp1, dinv, b1.reshape(1, -1), W2)
    acc2 = _edge_scatter(src2d, dst2d, hwp2)
    eps = jax.random.normal(jax.random.key(42), (N, LAT), jnp.float32)
    recon, cap, mu, logvar = _head(
        acc2, hwp2, dinv, b2.reshape(1, -1), Wmu, bmu.reshape(1, -1),
        Wlv, blv.reshape(1, -1), W3, b3.reshape(1, -1), W4, b4.reshape(1, -1),
        Wc, bc.reshape(1, -1), eps)
    return recon, cap, mu, logvar


# speedup vs baseline: 1.1491x; 1.1491x over previous
import functools

import jax
import jax.numpy as jnp
from jax import lax
from jax.experimental import pallas as pl
from jax.experimental.pallas import tpu as pltpu
from jax.experimental.pallas import tpu_sc as plsc

N = 10000
E = 320000
VOCAB = 1000
F = 128
HID = 128
LAT = 64

NC = 2
NS = 16
NW = NC * NS
EPT = E // NW
RPT = N // NS

_MESH = dict(core_axis_name="c", subcore_axis_name="s")


def _deg_partials(dst):
    @functools.partial(
        pl.kernel,
        out_type=jax.ShapeDtypeStruct((NW, N), jnp.float32),
        mesh=plsc.VectorSubcoreMesh(**_MESH),
        scratch_types=[
            pltpu.VMEM((N,), jnp.float32),
            pltpu.VMEM((2000,), jnp.int32),
        ],
        compiler_params=pltpu.CompilerParams(needs_layout_passes=False),
    )
    def k(dst_hbm, out_hbm, hist, dbuf):
        wid = lax.axis_index("c") * NS + lax.axis_index("s")
        zeros16 = jnp.zeros((16,), jnp.float32)
        ones16 = jnp.full((16,), 1.0, jnp.float32)

        def zbody(i, c):
            hist[pl.ds(i * 16, 16)] = zeros16
            return c

        lax.fori_loop(0, N // 16, zbody, 0)

        ebase = wid * EPT

        def cbody(cc, c):
            b = pl.multiple_of(ebase + cc * 2000, 8)
            pltpu.sync_copy(dst_hbm.at[pl.ds(b, 2000)], dbuf)

            def ibody(j, c2):
                idx = dbuf[pl.ds(j * 16, 16)]
                plsc.addupdate_scatter(hist, [idx], ones16)
                return c2

            lax.fori_loop(0, 2000 // 16, ibody, 0)
            return c

        lax.fori_loop(0, EPT // 2000, cbody, 0)
        pltpu.sync_copy(hist, out_hbm.at[wid])

    return k(dst)


def _emb_gather(x, table):
    @functools.partial(
        pl.kernel,
        out_type=jax.ShapeDtypeStruct((N, F), jnp.float32),
        mesh=plsc.VectorSubcoreMesh(**_MESH),
        scratch_types=[
            pltpu.VMEM((312,), jnp.int32),
            pltpu.VMEM((104, F), jnp.float32),
            pltpu.VMEM((16,), jnp.int32),
            pltpu.VMEM((16, F), jnp.float32),
            pltpu.SemaphoreType.DMA,
        ],
    )
    def k(x_hbm, tab_hbm, out_hbm, xbuf, rows, xbuf2, rows2, sem):
        wid = lax.axis_index("c") * NS + lax.axis_index("s")
        base = pl.multiple_of(wid * 312, 8)
        pltpu.sync_copy(x_hbm.at[pl.ds(base, 312)], xbuf)
        for kk in range(3):
            pltpu.async_copy(
                tab_hbm.at[xbuf.at[pl.ds(kk * 104, 104)]], rows, sem
            ).wait()
            pltpu.sync_copy(rows, out_hbm.at[pl.ds(base + kk * 104, 104)])

        @pl.when(wid == 0)
        def _tail():
            pltpu.sync_copy(x_hbm.at[pl.ds(312 * NW, 16)], xbuf2)
            pltpu.async_copy(tab_hbm.at[xbuf2], rows2, sem).wait()
            pltpu.sync_copy(rows2, out_hbm.at[pl.ds(312 * NW, 16)])

    return k(x, table)


NPAD = N + 128
CPT = 80
NGRP = CPT // 8


def _edge_scatter(src2d, dst2d, hwp):
    @functools.partial(
        pl.kernel,
        out_type=jax.ShapeDtypeStruct((NC, N, F), jnp.float32),
        mesh=plsc.VectorSubcoreMesh(**_MESH),
        scratch_types=[
            [pltpu.VMEM((128,), jnp.int32)] * 2,
            [pltpu.VMEM((128,), jnp.int32)] * 2,
            pltpu.VMEM((128, F), jnp.float32),
            pltpu.VMEM((128, F), jnp.float32),
            pltpu.VMEM_SHARED((NPAD, F), jnp.float32),
            [pltpu.SemaphoreType.DMA] * 2,
        ],
    )
    def k(src_hbm, dst_hbm, hwp_hbm, out_hbm,
          sbufs, dbufs, r0, r1, acc, gsems):
        cid = lax.axis_index("c")
        sid = lax.axis_index("s")
        wid = cid * NS + sid
        rows = [r0, r1]
        zeros16 = jnp.zeros((16,), jnp.float32)

        def zbody(i, c):
            for c8 in range(F // 16):
                r0[i, pl.ds(c8 * 16, 16)] = zeros16
            return c

        lax.fori_loop(0, 128, zbody, 0)

        chunks_a = [(0, 128), (128, 128), (256, 128), (384, 128), (512, 112)]
        chunks_b = [(0, 128), (128, 128), (256, 128), (384, 128), (512, 128)]

        @pl.when(sid != NS - 1)
        def _init_a():
            base = pl.multiple_of(sid * 624, 8)
            for off, sz in chunks_a:
                pltpu.sync_copy(r0.at[pl.ds(0, sz)],
                                acc.at[pl.ds(base + off, sz)])

        @pl.when(sid == NS - 1)
        def _init_b():
            for off, sz in chunks_b:
                pltpu.sync_copy(r0.at[pl.ds(0, sz)],
                                acc.at[pl.ds(9360 + off, sz)])

        @pl.when(sid == 0)
        def _init_trash():
            pltpu.sync_copy(r0, acc.at[pl.ds(N, NPAD - N)])

        plsc.subcore_barrier()

        gbase = wid * (CPT * 128)

        def load_idx(chunk, s):
            b = pl.multiple_of(gbase + chunk * 128, 8)
            pltpu.sync_copy(src_hbm.at[pl.ds(b, 128)], sbufs[s])
            pltpu.sync_copy(dst_hbm.at[pl.ds(b, 128)], dbufs[s])

        load_idx(0, 0)

        def pbody(p, c):
            g0 = pltpu.async_copy(hwp_hbm.at[sbufs[0]], rows[0], gsems[0])
            load_idx(2 * p + 1, 1)
            g1 = pltpu.async_copy(hwp_hbm.at[sbufs[1]], rows[1], gsems[1])
            g0.wait()
            pltpu.sync_copy(rows[0], acc.at[dbufs[0]], add=True)

            @pl.when(p != CPT // 2 - 1)
            def _nx():
                load_idx(2 * p + 2, 0)

            g1.wait()
            pltpu.sync_copy(rows[1], acc.at[dbufs[1]], add=True)
            return c

        lax.fori_loop(0, CPT // 2, pbody, 0)

        plsc.subcore_barrier()

        @pl.when(sid != NS - 1)
        def _wb_a():
            base = pl.multiple_of(sid * 624, 8)
            for off, sz in chunks_a:
                pltpu.sync_copy(acc.at[pl.ds(base + off, sz)],
                                out_hbm.at[cid, pl.ds(base + off, sz)])

        @pl.when(sid == NS - 1)
        def _wb_b():
            for off, sz in chunks_b:
                pltpu.sync_copy(acc.at[pl.ds(9360 + off, sz)],
                                out_hbm.at[cid, pl.ds(9360 + off, sz)])

    return k(src2d, dst2d, hwp)


def _prep(hist, emb, W1):
    def body(hist_ref, emb_ref, w1_ref, dinv_ref, t1_ref):
        ones = jnp.ones((NW, 1), jnp.float32)
        deg = lax.dot_general(
            hist_ref[...], ones, (((0,), (0,)), ((), ())),
            preferred_element_type=jnp.float32,
            precision=lax.Precision.HIGHEST,
        )
        dinv_ref[...] = lax.rsqrt(deg + 1.0)
        t1_ref[...] = jnp.dot(emb_ref[...], w1_ref[...],
                              preferred_element_type=jnp.float32)

    return pl.pallas_call(
        body,
        out_shape=(
            jax.ShapeDtypeStruct((N, 1), jnp.float32),
            jax.ShapeDtypeStruct((VOCAB, F), jnp.float32),
        ),
    )(hist, emb, W1)


_BLK = 1000
_NB = N // _BLK


def _scale(rows, dinv):
    def body(rows_ref, dinv_ref, out_ref):
        out_ref[...] = rows_ref[...] * dinv_ref[...]

    return pl.pallas_call(
        body,
        grid=(_NB,),
        in_specs=[
            pl.BlockSpec((_BLK, F), lambda i: (i, 0)),
            pl.BlockSpec((_BLK, 1), lambda i: (i, 0)),
        ],
        out_specs=pl.BlockSpec((_BLK, F), lambda i: (i, 0)),
        out_shape=jax.ShapeDtypeStruct((N, F), jnp.float32),
    )(rows, dinv)


def _combine(acc, hwp, dinv, b, W):
    def body(acc_ref, hwp_ref, dinv_ref, b_ref, w_ref, out_ref):
        s = acc_ref[0] + acc_ref[1] + hwp_ref[...]
        h = jnp.maximum(s * dinv_ref[...] + b_ref[...], 0.0)
        out_ref[...] = jnp.dot(h, w_ref[...],
                               preferred_element_type=jnp.float32) * dinv_ref[...]

    return pl.pallas_call(
        body,
        grid=(_NB,),
        in_specs=[
            pl.BlockSpec((NC, _BLK, F), lambda i: (0, i, 0)),
            pl.BlockSpec((_BLK, F), lambda i: (i, 0)),
            pl.BlockSpec((_BLK, 1), lambda i: (i, 0)),
            pl.BlockSpec((1, F), lambda i: (0, 0)),
            pl.BlockSpec((F, F), lambda i: (0, 0)),
        ],
        out_specs=pl.BlockSpec((_BLK, F), lambda i: (i, 0)),
        out_shape=jax.ShapeDtypeStruct((N, F), jnp.float32),
    )(acc, hwp, dinv, b, W)


def _head(acc, hwp, dinv, b2, Wmu, bmu, Wlv, blv, W3, b3, W4, b4, Wc, bc, eps):
    def body(acc_ref, hwp_ref, dinv_ref, b2_ref, wmu_ref, bmu_ref, wlv_ref,
             blv_ref, w3_ref, b3_ref, w4_ref, b4_ref, wc_ref, bc_ref, eps_ref,
             recon_ref, cap_ref, mu_ref, lv_ref):
        s = acc_ref[0] + acc_ref[1] + hwp_ref[...]
        h2 = jnp.maximum(s * dinv_ref[...] + b2_ref[...], 0.0)
        mu = jnp.dot(h2, wmu_ref[...], preferred_element_type=jnp.float32) + bmu_ref[...]
        lv = jnp.dot(h2, wlv_ref[...], preferred_element_type=jnp.float32) + blv_ref[...]
        mu_ref[...] = mu
        lv_ref[...] = lv
        z = mu + eps_ref[...] * jnp.exp(0.5 * lv)
        d = jnp.maximum(jnp.dot(z, w3_ref[...],
                                preferred_element_type=jnp.float32) + b3_ref[...], 0.0)
        recon_ref[...] = jnp.dot(d, w4_ref[...],
                                 preferred_element_type=jnp.float32) + b4_ref[...]
        logit = jnp.dot(z, wc_ref[...],
                        preferred_element_type=jnp.float32) + bc_ref[...]
        cap_ref[...] = jax.nn.sigmoid(logit)

    return pl.pallas_call(
        body,
        grid=(_NB,),
        in_specs=[
            pl.BlockSpec((NC, _BLK, F), lambda i: (0, i, 0)),
            pl.BlockSpec((_BLK, F), lambda i: (i, 0)),
            pl.BlockSpec((_BLK, 1), lambda i: (i, 0)),
            pl.BlockSpec((1, HID), lambda i: (0, 0)),
            pl.BlockSpec((HID, LAT), lambda i: (0, 0)),
            pl.BlockSpec((1, LAT), lambda i: (0, 0)),
            pl.BlockSpec((HID, LAT), lambda i: (0, 0)),
            pl.BlockSpec((1, LAT), lambda i: (0, 0)),
            pl.BlockSpec((LAT, HID), lambda i: (0, 0)),
            pl.BlockSpec((1, HID), lambda i: (0, 0)),
            pl.BlockSpec((HID, VOCAB), lambda i: (0, 0)),
            pl.BlockSpec((1, VOCAB), lambda i: (0, 0)),
            pl.BlockSpec((LAT, 1), lambda i: (0, 0)),
            pl.BlockSpec((1, 1), lambda i: (0, 0)),
            pl.BlockSpec((_BLK, LAT), lambda i: (i, 0)),
        ],
        out_specs=(
            pl.BlockSpec((_BLK, VOCAB), lambda i: (i, 0)),
            pl.BlockSpec((_BLK, 1), lambda i: (i, 0)),
            pl.BlockSpec((_BLK, LAT), lambda i: (i, 0)),
            pl.BlockSpec((_BLK, LAT), lambda i: (i, 0)),
        ),
        out_shape=(
            jax.ShapeDtypeStruct((N, VOCAB), jnp.float32),
            jax.ShapeDtypeStruct((N, 1), jnp.float32),
            jax.ShapeDtypeStruct((N, LAT), jnp.float32),
            jax.ShapeDtypeStruct((N, LAT), jnp.float32),
        ),
    )(acc, hwp, dinv, b2, Wmu, bmu, Wlv, blv, W3, b3, W4, b4, Wc, bc, eps)


def kernel(x, edge_index, emb, W1, b1, W2, b2, Wmu, bmu, Wlv, blv,
           W3, b3, W4, b4, Wc, bc):
    x = x.astype(jnp.int32)
    src = edge_index[0].astype(jnp.int32)
    dst = edge_index[1].astype(jnp.int32)
    npad = CPT * 128 - EPT
    src2d = jnp.pad(src.reshape(NW, EPT), ((0, 0), (0, npad))
                    ).reshape(NW * CPT * 128)
    padd = jnp.broadcast_to(N + (jnp.arange(npad, dtype=jnp.int32) % 128),
                            (NW, npad))
    dst2d = jnp.concatenate([dst.reshape(NW, EPT), padd],
                            axis=1).reshape(NW * CPT * 128)

    hist = _deg_partials(dst)
    dinv, T1 = _prep(hist, emb, W1)
    rows = _emb_gather(x, T1)
    hwp1 = _scale(rows, dinv)
    acc1 = _edge_scatter(src2d, dst2d, hwp1)
    hwp2 = _combine(acc1, hwp1, dinv, b1.reshape(1, -1), W2)
    acc2 = _edge_scatter(src2d, dst2d, hwp2)
    eps = jax.random.normal(jax.random.key(42), (N, LAT), jnp.float32)
    recon, cap, mu, logvar = _head(
        acc2, hwp2, dinv, b2.reshape(1, -1), Wmu, bmu.reshape(1, -1),
        Wlv, blv.reshape(1, -1), W3, b3.reshape(1, -1), W4, b4.reshape(1, -1),
        Wc, bc.reshape(1, -1), eps)
    return recon, cap, mu, logvar


# speedup vs baseline: 2.4973x; 2.1732x over previous
import functools

import jax
import jax.numpy as jnp
from jax import lax
from jax.experimental import pallas as pl
from jax.experimental.pallas import tpu as pltpu
from jax.experimental.pallas import tpu_sc as plsc

N = 10000
E = 320000
VOCAB = 1000
F = 128
HID = 128
LAT = 64

NC = 2
NS = 16
NW = NC * NS
EPT = E // NW
RPT = N // NS

_MESH = dict(core_axis_name="c", subcore_axis_name="s")


def _deg_partials(dst):
    @functools.partial(
        pl.kernel,
        out_type=jax.ShapeDtypeStruct((NW, N), jnp.float32),
        mesh=plsc.VectorSubcoreMesh(**_MESH),
        scratch_types=[
            pltpu.VMEM((N,), jnp.float32),
            pltpu.VMEM((2000,), jnp.int32),
        ],
        compiler_params=pltpu.CompilerParams(needs_layout_passes=False),
    )
    def k(dst_hbm, out_hbm, hist, dbuf):
        wid = lax.axis_index("c") * NS + lax.axis_index("s")
        zeros16 = jnp.zeros((16,), jnp.float32)
        ones16 = jnp.full((16,), 1.0, jnp.float32)

        def zbody(i, c):
            hist[pl.ds(i * 16, 16)] = zeros16
            return c

        lax.fori_loop(0, N // 16, zbody, 0)

        ebase = wid * EPT

        def cbody(cc, c):
            b = pl.multiple_of(ebase + cc * 2000, 8)
            pltpu.sync_copy(dst_hbm.at[pl.ds(b, 2000)], dbuf)

            def ibody(j, c2):
                idx = dbuf[pl.ds(j * 16, 16)]
                plsc.addupdate_scatter(hist, [idx], ones16)
                return c2

            lax.fori_loop(0, 2000 // 16, ibody, 0)
            return c

        lax.fori_loop(0, EPT // 2000, cbody, 0)
        pltpu.sync_copy(hist, out_hbm.at[wid])

    return k(dst)


def _emb_gather(x, table):
    @functools.partial(
        pl.kernel,
        out_type=jax.ShapeDtypeStruct((N, F), jnp.float32),
        mesh=plsc.VectorSubcoreMesh(**_MESH),
        scratch_types=[
            pltpu.VMEM((312,), jnp.int32),
            pltpu.VMEM((104, F), jnp.float32),
            pltpu.VMEM((16,), jnp.int32),
            pltpu.VMEM((16, F), jnp.float32),
            pltpu.SemaphoreType.DMA,
        ],
    )
    def k(x_hbm, tab_hbm, out_hbm, xbuf, rows, xbuf2, rows2, sem):
        wid = lax.axis_index("c") * NS + lax.axis_index("s")
        base = pl.multiple_of(wid * 312, 8)
        pltpu.sync_copy(x_hbm.at[pl.ds(base, 312)], xbuf)
        for kk in range(3):
            pltpu.async_copy(
                tab_hbm.at[xbuf.at[pl.ds(kk * 104, 104)]], rows, sem
            ).wait()
            pltpu.sync_copy(rows, out_hbm.at[pl.ds(base + kk * 104, 104)])

        @pl.when(wid == 0)
        def _tail():
            pltpu.sync_copy(x_hbm.at[pl.ds(312 * NW, 16)], xbuf2)
            pltpu.async_copy(tab_hbm.at[xbuf2], rows2, sem).wait()
            pltpu.sync_copy(rows2, out_hbm.at[pl.ds(312 * NW, 16)])

    return k(x, table)


NPAD = N + 128
CPT = 80
NGRP = CPT // 8


def _edge_scatter(src2d, dst2d, hwp):
    @functools.partial(
        pl.kernel,
        out_type=jax.ShapeDtypeStruct((NC, N, F), jnp.float32),
        mesh=plsc.VectorSubcoreMesh(**_MESH),
        scratch_types=[
            [pltpu.VMEM((128,), jnp.int32)] * 2,
            [pltpu.VMEM((128,), jnp.int32)] * 2,
            pltpu.VMEM((128, F), jnp.float32),
            pltpu.VMEM((128, F), jnp.float32),
            pltpu.VMEM_SHARED((NPAD, F), jnp.float32),
            [pltpu.SemaphoreType.DMA] * 2,
        ],
    )
    def k(src_hbm, dst_hbm, hwp_hbm, out_hbm,
          sbufs, dbufs, r0, r1, acc, gsems):
        cid = lax.axis_index("c")
        sid = lax.axis_index("s")
        wid = cid * NS + sid
        rows = [r0, r1]
        zeros16 = jnp.zeros((16,), jnp.float32)

        def zbody(i, c):
            for c8 in range(F // 16):
                r0[i, pl.ds(c8 * 16, 16)] = zeros16
            return c

        lax.fori_loop(0, 128, zbody, 0)

        chunks_a = [(0, 128), (128, 128), (256, 128), (384, 128), (512, 112)]
        chunks_b = [(0, 128), (128, 128), (256, 128), (384, 128), (512, 128)]

        @pl.when(sid != NS - 1)
        def _init_a():
            base = pl.multiple_of(sid * 624, 8)
            for off, sz in chunks_a:
                pltpu.sync_copy(r0.at[pl.ds(0, sz)],
                                acc.at[pl.ds(base + off, sz)])

        @pl.when(sid == NS - 1)
        def _init_b():
            for off, sz in chunks_b:
                pltpu.sync_copy(r0.at[pl.ds(0, sz)],
                                acc.at[pl.ds(9360 + off, sz)])

        @pl.when(sid == 0)
        def _init_trash():
            pltpu.sync_copy(r0, acc.at[pl.ds(N, NPAD - N)])

        plsc.subcore_barrier()

        gbase = wid * (CPT * 128)

        def load_idx(chunk, s):
            b = pl.multiple_of(gbase + chunk * 128, 8)
            pltpu.sync_copy(src_hbm.at[pl.ds(b, 128)], sbufs[s])
            pltpu.sync_copy(dst_hbm.at[pl.ds(b, 128)], dbufs[s])

        load_idx(0, 0)

        def pbody(p, c):
            g0 = pltpu.async_copy(hwp_hbm.at[sbufs[0]], rows[0], gsems[0])
            load_idx(2 * p + 1, 1)
            g1 = pltpu.async_copy(hwp_hbm.at[sbufs[1]], rows[1], gsems[1])
            g0.wait()
            pltpu.sync_copy(rows[0], acc.at[dbufs[0]], add=True)

            @pl.when(p != CPT // 2 - 1)
            def _nx():
                load_idx(2 * p + 2, 0)

            g1.wait()
            pltpu.sync_copy(rows[1], acc.at[dbufs[1]], add=True)
            return c

        lax.fori_loop(0, CPT // 2, pbody, 0)

        plsc.subcore_barrier()

        @pl.when(sid != NS - 1)
        def _wb_a():
            base = pl.multiple_of(sid * 624, 8)
            for off, sz in chunks_a:
                pltpu.sync_copy(acc.at[pl.ds(base + off, sz)],
                                out_hbm.at[cid, pl.ds(base + off, sz)])

        @pl.when(sid == NS - 1)
        def _wb_b():
            for off, sz in chunks_b:
                pltpu.sync_copy(acc.at[pl.ds(9360 + off, sz)],
                                out_hbm.at[cid, pl.ds(9360 + off, sz)])

    return k(src2d, dst2d, hwp)


def _prep(hist, emb, W1):
    def body(hist_ref, emb_ref, w1_ref, dinv_ref, t1_ref):
        ones = jnp.ones((NW, 1), jnp.float32)
        deg = lax.dot_general(
            hist_ref[...], ones, (((0,), (0,)), ((), ())),
            preferred_element_type=jnp.float32,
            precision=lax.Precision.HIGHEST,
        )
        dinv_ref[...] = lax.rsqrt(deg + 1.0)
        t1_ref[...] = jnp.dot(emb_ref[...], w1_ref[...],
                              preferred_element_type=jnp.float32)

    return pl.pallas_call(
        body,
        out_shape=(
            jax.ShapeDtypeStruct((N, 1), jnp.float32),
            jax.ShapeDtypeStruct((VOCAB, F), jnp.float32),
        ),
    )(hist, emb, W1)


_BLK = 1000
_NB = N // _BLK


def _scale(rows, dinv):
    def body(rows_ref, dinv_ref, out_ref):
        out_ref[...] = rows_ref[...] * dinv_ref[...]

    return pl.pallas_call(
        body,
        grid=(_NB,),
        in_specs=[
            pl.BlockSpec((_BLK, F), lambda i: (i, 0)),
            pl.BlockSpec((_BLK, 1), lambda i: (i, 0)),
        ],
        out_specs=pl.BlockSpec((_BLK, F), lambda i: (i, 0)),
        out_shape=jax.ShapeDtypeStruct((N, F), jnp.float32),
    )(rows, dinv)


def _combine(acc, hwp, dinv, b, W):
    def body(acc_ref, hwp_ref, dinv_ref, b_ref, w_ref, out_ref):
        s = acc_ref[0] + acc_ref[1] + hwp_ref[...]
        h = jnp.maximum(s * dinv_ref[...] + b_ref[...], 0.0)
        out_ref[...] = jnp.dot(h, w_ref[...],
                               preferred_element_type=jnp.float32) * dinv_ref[...]

    return pl.pallas_call(
        body,
        grid=(_NB,),
        in_specs=[
            pl.BlockSpec((NC, _BLK, F), lambda i: (0, i, 0)),
            pl.BlockSpec((_BLK, F), lambda i: (i, 0)),
            pl.BlockSpec((_BLK, 1), lambda i: (i, 0)),
            pl.BlockSpec((1, F), lambda i: (0, 0)),
            pl.BlockSpec((F, F), lambda i: (0, 0)),
        ],
        out_specs=pl.BlockSpec((_BLK, F), lambda i: (i, 0)),
        out_shape=jax.ShapeDtypeStruct((N, F), jnp.float32),
    )(acc, hwp, dinv, b, W)


def _head(acc, hwp, dinv, b2, Wmu, bmu, Wlv, blv, W3, b3, W4, b4, Wc, bc, eps):
    def body(acc_ref, hwp_ref, dinv_ref, b2_ref, wmu_ref, bmu_ref, wlv_ref,
             blv_ref, w3_ref, b3_ref, w4_ref, b4_ref, wc_ref, bc_ref, eps_ref,
             recon_ref, cap_ref, mu_ref, lv_ref):
        s = acc_ref[0] + acc_ref[1] + hwp_ref[...]
        h2 = jnp.maximum(s * dinv_ref[...] + b2_ref[...], 0.0)
        mu = jnp.dot(h2, wmu_ref[...], preferred_element_type=jnp.float32) + bmu_ref[...]
        lv = jnp.dot(h2, wlv_ref[...], preferred_element_type=jnp.float32) + blv_ref[...]
        mu_ref[...] = mu
        lv_ref[...] = lv
        z = mu + eps_ref[...] * jnp.exp(0.5 * lv)
        d = jnp.maximum(jnp.dot(z, w3_ref[...],
                                preferred_element_type=jnp.float32) + b3_ref[...], 0.0)
        recon_ref[...] = jnp.dot(d, w4_ref[...],
                                 preferred_element_type=jnp.float32) + b4_ref[...]
        logit = jnp.dot(z, wc_ref[...],
                        preferred_element_type=jnp.float32) + bc_ref[...]
        cap_ref[...] = jax.nn.sigmoid(logit)

    return pl.pallas_call(
        body,
        grid=(_NB,),
        in_specs=[
            pl.BlockSpec((NC, _BLK, F), lambda i: (0, i, 0)),
            pl.BlockSpec((_BLK, F), lambda i: (i, 0)),
            pl.BlockSpec((_BLK, 1), lambda i: (i, 0)),
            pl.BlockSpec((1, HID), lambda i: (0, 0)),
            pl.BlockSpec((HID, LAT), lambda i: (0, 0)),
            pl.BlockSpec((1, LAT), lambda i: (0, 0)),
            pl.BlockSpec((HID, LAT), lambda i: (0, 0)),
            pl.BlockSpec((1, LAT), lambda i: (0, 0)),
            pl.BlockSpec((LAT, HID), lambda i: (0, 0)),
            pl.BlockSpec((1, HID), lambda i: (0, 0)),
            pl.BlockSpec((HID, VOCAB), lambda i: (0, 0)),
            pl.BlockSpec((1, VOCAB), lambda i: (0, 0)),
            pl.BlockSpec((LAT, 1), lambda i: (0, 0)),
            pl.BlockSpec((1, 1), lambda i: (0, 0)),
            pl.BlockSpec((_BLK, LAT), lambda i: (i, 0)),
        ],
        out_specs=(
            pl.BlockSpec((_BLK, VOCAB), lambda i: (i, 0)),
            pl.BlockSpec((_BLK, 1), lambda i: (i, 0)),
            pl.BlockSpec((_BLK, LAT), lambda i: (i, 0)),
            pl.BlockSpec((_BLK, LAT), lambda i: (i, 0)),
        ),
        out_shape=(
            jax.ShapeDtypeStruct((N, VOCAB), jnp.float32),
            jax.ShapeDtypeStruct((N, 1), jnp.float32),
            jax.ShapeDtypeStruct((N, LAT), jnp.float32),
            jax.ShapeDtypeStruct((N, LAT), jnp.float32),
        ),
    )(acc, hwp, dinv, b2, Wmu, bmu, Wlv, blv, W3, b3, W4, b4, Wc, bc, eps)


def kernel(x, edge_index, emb, W1, b1, W2, b2, Wmu, bmu, Wlv, blv,
           W3, b3, W4, b4, Wc, bc):
    x = x.astype(jnp.int32)
    src = edge_index[0].astype(jnp.int32)
    dst = edge_index[1].astype(jnp.int32)
    npad = CPT * 128 - EPT
    spread = jnp.arange(npad, dtype=jnp.int32) % 128
    pads = jnp.broadcast_to(spread, (NW, npad))
    padd = jnp.broadcast_to(N + spread, (NW, npad))
    src2d = jnp.concatenate([src.reshape(NW, EPT), pads],
                            axis=1).reshape(NW * CPT * 128)
    dst2d = jnp.concatenate([dst.reshape(NW, EPT), padd],
                            axis=1).reshape(NW * CPT * 128)

    hist = _deg_partials(dst)
    dinv, T1 = _prep(hist, emb, W1)
    rows = _emb_gather(x, T1)
    hwp1 = _scale(rows, dinv)
    acc1 = _edge_scatter(src2d, dst2d, hwp1)
    hwp2 = _combine(acc1, hwp1, dinv, b1.reshape(1, -1), W2)
    acc2 = _edge_scatter(src2d, dst2d, hwp2)
    eps = jax.random.normal(jax.random.key(42), (N, LAT), jnp.float32)
    recon, cap, mu, logvar = _head(
        acc2, hwp2, dinv, b2.reshape(1, -1), Wmu, bmu.reshape(1, -1),
        Wlv, blv.reshape(1, -1), W3, b3.reshape(1, -1), W4, b4.reshape(1, -1),
        Wc, bc.reshape(1, -1), eps)
    return recon, cap, mu, logvar


# speedup vs baseline: 3.1847x; 1.2753x over previous
import functools

import jax
import jax.numpy as jnp
from jax import lax
from jax.experimental import pallas as pl
from jax.experimental.pallas import tpu as pltpu
from jax.experimental.pallas import tpu_sc as plsc

N = 10000
E = 320000
VOCAB = 1000
F = 128
HID = 128
LAT = 64

NC = 2
NS = 16
NW = NC * NS
EPT = E // NW
RPT = N // NS

_MESH = dict(core_axis_name="c", subcore_axis_name="s")


def _deg_partials(dst):
    @functools.partial(
        pl.kernel,
        out_type=jax.ShapeDtypeStruct((NW, N), jnp.float32),
        mesh=plsc.VectorSubcoreMesh(**_MESH),
        scratch_types=[
            pltpu.VMEM((N,), jnp.float32),
            pltpu.VMEM((2000,), jnp.int32),
        ],
        compiler_params=pltpu.CompilerParams(needs_layout_passes=False),
    )
    def k(dst_hbm, out_hbm, hist, dbuf):
        wid = lax.axis_index("c") * NS + lax.axis_index("s")
        zeros16 = jnp.zeros((16,), jnp.float32)
        ones16 = jnp.full((16,), 1.0, jnp.float32)

        def zbody(i, c):
            hist[pl.ds(i * 16, 16)] = zeros16
            return c

        lax.fori_loop(0, N // 16, zbody, 0)

        ebase = wid * EPT

        def cbody(cc, c):
            b = pl.multiple_of(ebase + cc * 2000, 8)
            pltpu.sync_copy(dst_hbm.at[pl.ds(b, 2000)], dbuf)

            def ibody(j, c2):
                idx = dbuf[pl.ds(j * 16, 16)]
                plsc.addupdate_scatter(hist, [idx], ones16)
                return c2

            lax.fori_loop(0, 2000 // 16, ibody, 0)
            return c

        lax.fori_loop(0, EPT // 2000, cbody, 0)
        pltpu.sync_copy(hist, out_hbm.at[wid])

    return k(dst)


def _emb_gather(x, table):
    @functools.partial(
        pl.kernel,
        out_type=jax.ShapeDtypeStruct((N, F), jnp.float32),
        mesh=plsc.VectorSubcoreMesh(**_MESH),
        scratch_types=[
            pltpu.VMEM((312,), jnp.int32),
            pltpu.VMEM((104, F), jnp.float32),
            pltpu.VMEM((16,), jnp.int32),
            pltpu.VMEM((16, F), jnp.float32),
            pltpu.SemaphoreType.DMA,
        ],
    )
    def k(x_hbm, tab_hbm, out_hbm, xbuf, rows, xbuf2, rows2, sem):
        wid = lax.axis_index("c") * NS + lax.axis_index("s")
        base = pl.multiple_of(wid * 312, 8)
        pltpu.sync_copy(x_hbm.at[pl.ds(base, 312)], xbuf)
        for kk in range(3):
            pltpu.async_copy(
                tab_hbm.at[xbuf.at[pl.ds(kk * 104, 104)]], rows, sem
            ).wait()
            pltpu.sync_copy(rows, out_hbm.at[pl.ds(base + kk * 104, 104)])

        @pl.when(wid == 0)
        def _tail():
            pltpu.sync_copy(x_hbm.at[pl.ds(312 * NW, 16)], xbuf2)
            pltpu.async_copy(tab_hbm.at[xbuf2], rows2, sem).wait()
            pltpu.sync_copy(rows2, out_hbm.at[pl.ds(312 * NW, 16)])

    return k(x, table)


NPAD = N + 128
CPT = 80
NGRP = CPT // 8


def _edge_scatter(src2d, dst2d, hwp):
    @functools.partial(
        pl.kernel,
        out_type=jax.ShapeDtypeStruct((NC, N, F), jnp.float32),
        mesh=plsc.VectorSubcoreMesh(**_MESH),
        scratch_types=[
            [pltpu.VMEM((128,), jnp.int32)] * 8,
            [pltpu.VMEM((128,), jnp.int32)] * 8,
            pltpu.VMEM((128, F), jnp.float32),
            pltpu.VMEM((128, F), jnp.float32),
            pltpu.VMEM_SHARED((NPAD, F), jnp.float32),
            [pltpu.SemaphoreType.DMA] * 2,
            [pltpu.SemaphoreType.DMA] * 2,
            [pltpu.SemaphoreType.DMA] * 8,
        ],
    )
    def k(src_hbm, dst_hbm, hwp_hbm, out_hbm,
          sbufs, dbufs, r0, r1, acc, gsems, ssems, isems):
        cid = lax.axis_index("c")
        sid = lax.axis_index("s")
        wid = cid * NS + sid
        rows = [r0, r1]
        zeros16 = jnp.zeros((16,), jnp.float32)

        def zbody(i, c):
            for c8 in range(F // 16):
                r0[i, pl.ds(c8 * 16, 16)] = zeros16
            return c

        lax.fori_loop(0, 128, zbody, 0)

        chunks_a = [(0, 128), (128, 128), (256, 128), (384, 128), (512, 112)]
        chunks_b = [(0, 128), (128, 128), (256, 128), (384, 128), (512, 128)]

        @pl.when(sid != NS - 1)
        def _init_a():
            base = pl.multiple_of(sid * 624, 8)
            for off, sz in chunks_a:
                pltpu.sync_copy(r0.at[pl.ds(0, sz)],
                                acc.at[pl.ds(base + off, sz)])

        @pl.when(sid == NS - 1)
        def _init_b():
            for off, sz in chunks_b:
                pltpu.sync_copy(r0.at[pl.ds(0, sz)],
                                acc.at[pl.ds(9360 + off, sz)])

        @pl.when(sid == 0)
        def _init_trash():
            pltpu.sync_copy(r0, acc.at[pl.ds(N, NPAD - N)])

        plsc.subcore_barrier()

        gbase = wid * (CPT * 128)

        NB8 = CPT // 8

        def idx_load(slot, chunk):
            b = pl.multiple_of(gbase + chunk * 128, 8)
            pltpu.async_copy(src_hbm.at[pl.ds(b, 128)], sbufs[slot],
                             isems[slot])
            pltpu.async_copy(dst_hbm.at[pl.ds(b, 128)], dbufs[slot],
                             isems[slot])

        def idx_wait(slot):
            pltpu.make_async_copy(src_hbm.at[pl.ds(0, 128)], sbufs[slot],
                                  isems[slot]).wait()
            pltpu.make_async_copy(src_hbm.at[pl.ds(0, 128)], dbufs[slot],
                                  isems[slot]).wait()

        def drain_scatter(s):
            pltpu.make_async_copy(hwp_hbm.at[pl.ds(0, 128)], rows[s],
                                  ssems[s]).wait()

        def wait_gather(s):
            pltpu.make_async_copy(hwp_hbm.at[pl.ds(0, 128)], rows[s],
                                  gsems[s]).wait()

        for j in range(6):
            idx_load(j, j)

        def body(r, c):
            gd = [None, None]
            for j in range(8):
                s = j % 2
                ps = (j + 1) % 2
                pslot = (j - 2) % 8

                if j < 2:
                    @pl.when(r > 0)
                    def _d(s=s):
                        drain_scatter(s)

                    idx_load(pslot, 8 * r + j + 6)
                else:
                    drain_scatter(s)

                    @pl.when(r < NB8 - 1)
                    def _p2(pslot=pslot, r=r, j=j):
                        idx_load(pslot, 8 * r + j + 6)

                idx_wait(j)
                gd[s] = pltpu.async_copy(hwp_hbm.at[sbufs[j]], rows[s],
                                         gsems[s])
                if j == 0:
                    @pl.when(r > 0)
                    def _s0(ps=ps):
                        wait_gather(ps)
                        pltpu.async_copy(rows[ps], acc.at[dbufs[7]],
                                         ssems[ps], add=True)
                else:
                    gd[ps].wait()
                    pltpu.async_copy(rows[ps], acc.at[dbufs[j - 1]],
                                     ssems[ps], add=True)
            return c

        lax.fori_loop(0, NB8, body, 0)

        wait_gather(1)
        pltpu.async_copy(rows[1], acc.at[dbufs[7]], ssems[1], add=True)
        drain_scatter(0)
        drain_scatter(1)

        plsc.subcore_barrier()

        @pl.when(sid != NS - 1)
        def _wb_a():
            base = pl.multiple_of(sid * 624, 8)
            for off, sz in chunks_a:
                pltpu.sync_copy(acc.at[pl.ds(base + off, sz)],
                                out_hbm.at[cid, pl.ds(base + off, sz)])

        @pl.when(sid == NS - 1)
        def _wb_b():
            for off, sz in chunks_b:
                pltpu.sync_copy(acc.at[pl.ds(9360 + off, sz)],
                                out_hbm.at[cid, pl.ds(9360 + off, sz)])

    return k(src2d, dst2d, hwp)


def _prep(hist, emb, W1):
    def body(hist_ref, emb_ref, w1_ref, dinv_ref, t1_ref):
        ones = jnp.ones((NW, 1), jnp.float32)
        deg = lax.dot_general(
            hist_ref[...], ones, (((0,), (0,)), ((), ())),
            preferred_element_type=jnp.float32,
            precision=lax.Precision.HIGHEST,
        )
        dinv_ref[...] = lax.rsqrt(deg + 1.0)
        t1_ref[...] = jnp.dot(emb_ref[...], w1_ref[...],
                              preferred_element_type=jnp.float32)

    return pl.pallas_call(
        body,
        out_shape=(
            jax.ShapeDtypeStruct((N, 1), jnp.float32),
            jax.ShapeDtypeStruct((VOCAB, F), jnp.float32),
        ),
    )(hist, emb, W1)


_BLK = 1000
_NB = N // _BLK


def _scale(rows, dinv):
    def body(rows_ref, dinv_ref, out_ref):
        out_ref[...] = rows_ref[...] * dinv_ref[...]

    return pl.pallas_call(
        body,
        grid=(_NB,),
        in_specs=[
            pl.BlockSpec((_BLK, F), lambda i: (i, 0)),
            pl.BlockSpec((_BLK, 1), lambda i: (i, 0)),
        ],
        out_specs=pl.BlockSpec((_BLK, F), lambda i: (i, 0)),
        out_shape=jax.ShapeDtypeStruct((N, F), jnp.float32),
    )(rows, dinv)


def _combine(acc, hwp, dinv, b, W):
    def body(acc_ref, hwp_ref, dinv_ref, b_ref, w_ref, out_ref):
        s = acc_ref[0] + acc_ref[1] + hwp_ref[...]
        h = jnp.maximum(s * dinv_ref[...] + b_ref[...], 0.0)
        out_ref[...] = jnp.dot(h, w_ref[...],
                               preferred_element_type=jnp.float32) * dinv_ref[...]

    return pl.pallas_call(
        body,
        grid=(_NB,),
        in_specs=[
            pl.BlockSpec((NC, _BLK, F), lambda i: (0, i, 0)),
            pl.BlockSpec((_BLK, F), lambda i: (i, 0)),
            pl.BlockSpec((_BLK, 1), lambda i: (i, 0)),
            pl.BlockSpec((1, F), lambda i: (0, 0)),
            pl.BlockSpec((F, F), lambda i: (0, 0)),
        ],
        out_specs=pl.BlockSpec((_BLK, F), lambda i: (i, 0)),
        out_shape=jax.ShapeDtypeStruct((N, F), jnp.float32),
    )(acc, hwp, dinv, b, W)


def _head(acc, hwp, dinv, b2, Wmu, bmu, Wlv, blv, W3, b3, W4, b4, Wc, bc, eps):
    def body(acc_ref, hwp_ref, dinv_ref, b2_ref, wmu_ref, bmu_ref, wlv_ref,
             blv_ref, w3_ref, b3_ref, w4_ref, b4_ref, wc_ref, bc_ref, eps_ref,
             recon_ref, cap_ref, mu_ref, lv_ref):
        s = acc_ref[0] + acc_ref[1] + hwp_ref[...]
        h2 = jnp.maximum(s * dinv_ref[...] + b2_ref[...], 0.0)
        mu = jnp.dot(h2, wmu_ref[...], preferred_element_type=jnp.float32) + bmu_ref[...]
        lv = jnp.dot(h2, wlv_ref[...], preferred_element_type=jnp.float32) + blv_ref[...]
        mu_ref[...] = mu
        lv_ref[...] = lv
        z = mu + eps_ref[...] * jnp.exp(0.5 * lv)
        d = jnp.maximum(jnp.dot(z, w3_ref[...],
                                preferred_element_type=jnp.float32) + b3_ref[...], 0.0)
        recon_ref[...] = jnp.dot(d, w4_ref[...],
                                 preferred_element_type=jnp.float32) + b4_ref[...]
        logit = jnp.dot(z, wc_ref[...],
                        preferred_element_type=jnp.float32) + bc_ref[...]
        cap_ref[...] = jax.nn.sigmoid(logit)

    return pl.pallas_call(
        body,
        grid=(_NB,),
        in_specs=[
            pl.BlockSpec((NC, _BLK, F), lambda i: (0, i, 0)),
            pl.BlockSpec((_BLK, F), lambda i: (i, 0)),
            pl.BlockSpec((_BLK, 1), lambda i: (i, 0)),
            pl.BlockSpec((1, HID), lambda i: (0, 0)),
            pl.BlockSpec((HID, LAT), lambda i: (0, 0)),
            pl.BlockSpec((1, LAT), lambda i: (0, 0)),
            pl.BlockSpec((HID, LAT), lambda i: (0, 0)),
            pl.BlockSpec((1, LAT), lambda i: (0, 0)),
            pl.BlockSpec((LAT, HID), lambda i: (0, 0)),
            pl.BlockSpec((1, HID), lambda i: (0, 0)),
            pl.BlockSpec((HID, VOCAB), lambda i: (0, 0)),
            pl.BlockSpec((1, VOCAB), lambda i: (0, 0)),
            pl.BlockSpec((LAT, 1), lambda i: (0, 0)),
            pl.BlockSpec((1, 1), lambda i: (0, 0)),
            pl.BlockSpec((_BLK, LAT), lambda i: (i, 0)),
        ],
        out_specs=(
            pl.BlockSpec((_BLK, VOCAB), lambda i: (i, 0)),
            pl.BlockSpec((_BLK, 1), lambda i: (i, 0)),
            pl.BlockSpec((_BLK, LAT), lambda i: (i, 0)),
            pl.BlockSpec((_BLK, LAT), lambda i: (i, 0)),
        ),
        out_shape=(
            jax.ShapeDtypeStruct((N, VOCAB), jnp.float32),
            jax.ShapeDtypeStruct((N, 1), jnp.float32),
            jax.ShapeDtypeStruct((N, LAT), jnp.float32),
            jax.ShapeDtypeStruct((N, LAT), jnp.float32),
        ),
    )(acc, hwp, dinv, b2, Wmu, bmu, Wlv, blv, W3, b3, W4, b4, Wc, bc, eps)


def kernel(x, edge_index, emb, W1, b1, W2, b2, Wmu, bmu, Wlv, blv,
           W3, b3, W4, b4, Wc, bc):
    x = x.astype(jnp.int32)
    src = edge_index[0].astype(jnp.int32)
    dst = edge_index[1].astype(jnp.int32)
    npad = CPT * 128 - EPT
    spread = jnp.arange(npad, dtype=jnp.int32) % 128
    pads = jnp.broadcast_to(spread, (NW, npad))
    padd = jnp.broadcast_to(N + spread, (NW, npad))
    src2d = jnp.concatenate([src.reshape(NW, EPT), pads],
                            axis=1).reshape(NW * CPT * 128)
    dst2d = jnp.concatenate([dst.reshape(NW, EPT), padd],
                            axis=1).reshape(NW * CPT * 128)

    hist = _deg_partials(dst)
    dinv, T1 = _prep(hist, emb, W1)
    rows = _emb_gather(x, T1)
    hwp1 = _scale(rows, dinv)
    acc1 = _edge_scatter(src2d, dst2d, hwp1)
    hwp2 = _combine(acc1, hwp1, dinv, b1.reshape(1, -1), W2)
    acc2 = _edge_scatter(src2d, dst2d, hwp2)
    eps = jax.random.normal(jax.random.key(42), (N, LAT), jnp.float32)
    recon, cap, mu, logvar = _head(
        acc2, hwp2, dinv, b2.reshape(1, -1), Wmu, bmu.reshape(1, -1),
        Wlv, blv.reshape(1, -1), W3, b3.reshape(1, -1), W4, b4.reshape(1, -1),
        Wc, bc.reshape(1, -1), eps)
    return recon, cap, mu, logvar


# speedup vs baseline: 3.2979x; 1.0356x over previous
import functools

import jax
import jax.numpy as jnp
from jax import lax
from jax.experimental import pallas as pl
from jax.experimental.pallas import tpu as pltpu
from jax.experimental.pallas import tpu_sc as plsc

N = 10000
E = 320000
VOCAB = 1000
F = 128
HID = 128
LAT = 64

NC = 2
NS = 16
NW = NC * NS
EPT = E // NW
RPT = N // NS

_MESH = dict(core_axis_name="c", subcore_axis_name="s")


def _deg_gather(dst, x, table):
    @functools.partial(
        pl.kernel,
        out_type=(jax.ShapeDtypeStruct((NW, N), jnp.float32),
                  jax.ShapeDtypeStruct((N, F), jnp.float32)),
        mesh=plsc.VectorSubcoreMesh(**_MESH),
        scratch_types=[
            pltpu.VMEM((N,), jnp.float32),
            pltpu.VMEM((2000,), jnp.int32),
            pltpu.VMEM((312,), jnp.int32),
            pltpu.VMEM((312, F), jnp.float32),
            pltpu.VMEM((16,), jnp.int32),
            pltpu.VMEM((16, F), jnp.float32),
            pltpu.SemaphoreType.DMA,
        ],
        compiler_params=pltpu.CompilerParams(needs_layout_passes=False),
    )
    def k(dst_hbm, x_hbm, tab_hbm, out_hbm, rows_hbm,
          hist, dbuf, xbuf, rbuf, xbuf2, rbuf2, sem):
        wid = lax.axis_index("c") * NS + lax.axis_index("s")
        zeros16 = jnp.zeros((16,), jnp.float32)
        ones16 = jnp.full((16,), 1.0, jnp.float32)

        base = pl.multiple_of(wid * 312, 8)
        pltpu.sync_copy(x_hbm.at[pl.ds(base, 312)], xbuf)
        descs = [pltpu.async_copy(
            tab_hbm.at[xbuf.at[pl.ds(kk * 104, 104)]],
            rbuf.at[pl.ds(kk * 104, 104)], sem) for kk in range(3)]

        def zbody(i, c):
            hist[pl.ds(i * 16, 16)] = zeros16
            return c

        lax.fori_loop(0, N // 16, zbody, 0)

        ebase = wid * EPT

        def cbody(cc, c):
            b = pl.multiple_of(ebase + cc * 2000, 8)
            pltpu.sync_copy(dst_hbm.at[pl.ds(b, 2000)], dbuf)

            def ibody(j, c2):
                idx = dbuf[pl.ds(j * 16, 16)]
                plsc.addupdate_scatter(hist, [idx], ones16)
                return c2

            lax.fori_loop(0, 2000 // 16, ibody, 0)
            return c

        lax.fori_loop(0, EPT // 2000, cbody, 0)
        pltpu.sync_copy(hist, out_hbm.at[wid])

        for kk in range(3):
            descs[kk].wait()
        pltpu.sync_copy(rbuf, rows_hbm.at[pl.ds(base, 312)])

        @pl.when(wid == 0)
        def _tail():
            pltpu.sync_copy(x_hbm.at[pl.ds(312 * NW, 16)], xbuf2)
            pltpu.async_copy(tab_hbm.at[xbuf2], rbuf2, sem).wait()
            pltpu.sync_copy(rbuf2, rows_hbm.at[pl.ds(312 * NW, 16)])

    return k(dst, x, table)


NPAD = N + 128
CPT = 80
NGRP = CPT // 8


def _edge_scatter(src2d, dst2d, hwp):
    @functools.partial(
        pl.kernel,
        out_type=jax.ShapeDtypeStruct((NC, N, F), jnp.float32),
        mesh=plsc.VectorSubcoreMesh(**_MESH),
        scratch_types=[
            [pltpu.VMEM((128,), jnp.int32)] * 8,
            [pltpu.VMEM((128,), jnp.int32)] * 8,
            pltpu.VMEM((128, F), jnp.float32),
            pltpu.VMEM((128, F), jnp.float32),
            pltpu.VMEM_SHARED((NPAD, F), jnp.float32),
            [pltpu.SemaphoreType.DMA] * 2,
            [pltpu.SemaphoreType.DMA] * 2,
            [pltpu.SemaphoreType.DMA] * 8,
        ],
    )
    def k(src_hbm, dst_hbm, hwp_hbm, out_hbm,
          sbufs, dbufs, r0, r1, acc, gsems, ssems, isems):
        cid = lax.axis_index("c")
        sid = lax.axis_index("s")
        wid = cid * NS + sid
        rows = [r0, r1]
        zeros16 = jnp.zeros((16,), jnp.float32)

        def zbody(i, c):
            for c8 in range(F // 16):
                r0[i, pl.ds(c8 * 16, 16)] = zeros16
            return c

        lax.fori_loop(0, 128, zbody, 0)

        chunks_a = [(0, 128), (128, 128), (256, 128), (384, 128), (512, 112)]
        chunks_b = [(0, 128), (128, 128), (256, 128), (384, 128), (512, 128)]

        @pl.when(sid != NS - 1)
        def _init_a():
            base = pl.multiple_of(sid * 624, 8)
            for off, sz in chunks_a:
                pltpu.sync_copy(r0.at[pl.ds(0, sz)],
                                acc.at[pl.ds(base + off, sz)])

        @pl.when(sid == NS - 1)
        def _init_b():
            for off, sz in chunks_b:
                pltpu.sync_copy(r0.at[pl.ds(0, sz)],
                                acc.at[pl.ds(9360 + off, sz)])

        @pl.when(sid == 0)
        def _init_trash():
            pltpu.sync_copy(r0, acc.at[pl.ds(N, NPAD - N)])

        plsc.subcore_barrier()

        gbase = wid * (CPT * 128)

        NB8 = CPT // 8

        def idx_load(slot, chunk):
            b = pl.multiple_of(gbase + chunk * 128, 8)
            pltpu.async_copy(src_hbm.at[pl.ds(b, 128)], sbufs[slot],
                             isems[slot])
            pltpu.async_copy(dst_hbm.at[pl.ds(b, 128)], dbufs[slot],
                             isems[slot])

        def idx_wait(slot):
            pltpu.make_async_copy(src_hbm.at[pl.ds(0, 128)], sbufs[slot],
                                  isems[slot]).wait()
            pltpu.make_async_copy(src_hbm.at[pl.ds(0, 128)], dbufs[slot],
                                  isems[slot]).wait()

        def drain_scatter(s):
            pltpu.make_async_copy(hwp_hbm.at[pl.ds(0, 128)], rows[s],
                                  ssems[s]).wait()

        def wait_gather(s):
            pltpu.make_async_copy(hwp_hbm.at[pl.ds(0, 128)], rows[s],
                                  gsems[s]).wait()

        for j in range(6):
            idx_load(j, j)

        def body(r, c):
            gd = [None, None]
            for j in range(8):
                s = j % 2
                ps = (j + 1) % 2
                pslot = (j - 2) % 8

                if j < 2:
                    @pl.when(r > 0)
                    def _d(s=s):
                        drain_scatter(s)

                    idx_load(pslot, 8 * r + j + 6)
                else:
                    drain_scatter(s)

                    @pl.when(r < NB8 - 1)
                    def _p2(pslot=pslot, r=r, j=j):
                        idx_load(pslot, 8 * r + j + 6)

                idx_wait(j)
                gd[s] = pltpu.async_copy(hwp_hbm.at[sbufs[j]], rows[s],
                                         gsems[s])
                if j == 0:
                    @pl.when(r > 0)
                    def _s0(ps=ps):
                        wait_gather(ps)
                        pltpu.async_copy(rows[ps], acc.at[dbufs[7]],
                                         ssems[ps], add=True)
                else:
                    gd[ps].wait()
                    pltpu.async_copy(rows[ps], acc.at[dbufs[j - 1]],
                                     ssems[ps], add=True)
            return c

        lax.fori_loop(0, NB8, body, 0)

        wait_gather(1)
        pltpu.async_copy(rows[1], acc.at[dbufs[7]], ssems[1], add=True)
        drain_scatter(0)
        drain_scatter(1)

        plsc.subcore_barrier()

        @pl.when(sid != NS - 1)
        def _wb_a():
            base = pl.multiple_of(sid * 624, 8)
            for off, sz in chunks_a:
                pltpu.sync_copy(acc.at[pl.ds(base + off, sz)],
                                out_hbm.at[cid, pl.ds(base + off, sz)])

        @pl.when(sid == NS - 1)
        def _wb_b():
            for off, sz in chunks_b:
                pltpu.sync_copy(acc.at[pl.ds(9360 + off, sz)],
                                out_hbm.at[cid, pl.ds(9360 + off, sz)])

    return k(src2d, dst2d, hwp)


def _prep1(emb, W1):
    def body(emb_ref, w1_ref, t1_ref):
        t1_ref[...] = jnp.dot(emb_ref[...], w1_ref[...],
                              preferred_element_type=jnp.float32)

    return pl.pallas_call(
        body,
        out_shape=jax.ShapeDtypeStruct((VOCAB, F), jnp.float32),
    )(emb, W1)


_BLK = 1000
_NB = N // _BLK


def _prep2(hist, rows):
    def body(hist_ref, rows_ref, dinv_ref, hwp_ref):
        ones = jnp.ones((NW, 1), jnp.float32)
        deg = lax.dot_general(
            hist_ref[...], ones, (((0,), (0,)), ((), ())),
            preferred_element_type=jnp.float32,
            precision=lax.Precision.HIGHEST,
        )
        dinv = lax.rsqrt(deg + 1.0)
        dinv_ref[...] = dinv
        hwp_ref[...] = rows_ref[...] * dinv

    return pl.pallas_call(
        body,
        out_shape=(
            jax.ShapeDtypeStruct((N, 1), jnp.float32),
            jax.ShapeDtypeStruct((N, F), jnp.float32),
        ),
    )(hist, rows)


def _combine(acc, hwp, dinv, b, W):
    def body(acc_ref, hwp_ref, dinv_ref, b_ref, w_ref, out_ref):
        s = acc_ref[0] + acc_ref[1] + hwp_ref[...]
        h = jnp.maximum(s * dinv_ref[...] + b_ref[...], 0.0)
        out_ref[...] = jnp.dot(h, w_ref[...],
                               preferred_element_type=jnp.float32) * dinv_ref[...]

    return pl.pallas_call(
        body,
        grid=(_NB,),
        in_specs=[
            pl.BlockSpec((NC, _BLK, F), lambda i: (0, i, 0)),
            pl.BlockSpec((_BLK, F), lambda i: (i, 0)),
            pl.BlockSpec((_BLK, 1), lambda i: (i, 0)),
            pl.BlockSpec((1, F), lambda i: (0, 0)),
            pl.BlockSpec((F, F), lambda i: (0, 0)),
        ],
        out_specs=pl.BlockSpec((_BLK, F), lambda i: (i, 0)),
        out_shape=jax.ShapeDtypeStruct((N, F), jnp.float32),
    )(acc, hwp, dinv, b, W)


def _head(acc, hwp, dinv, b2, Wmu, bmu, Wlv, blv, W3, b3, W4, b4, Wc, bc, eps):
    def body(acc_ref, hwp_ref, dinv_ref, b2_ref, wmu_ref, bmu_ref, wlv_ref,
             blv_ref, w3_ref, b3_ref, w4_ref, b4_ref, wc_ref, bc_ref, eps_ref,
             recon_ref, cap_ref, mu_ref, lv_ref):
        s = acc_ref[0] + acc_ref[1] + hwp_ref[...]
        h2 = jnp.maximum(s * dinv_ref[...] + b2_ref[...], 0.0)
        mu = jnp.dot(h2, wmu_ref[...], preferred_element_type=jnp.float32) + bmu_ref[...]
        lv = jnp.dot(h2, wlv_ref[...], preferred_element_type=jnp.float32) + blv_ref[...]
        mu_ref[...] = mu
        lv_ref[...] = lv
        z = mu + eps_ref[...] * jnp.exp(0.5 * lv)
        d = jnp.maximum(jnp.dot(z, w3_ref[...],
                                preferred_element_type=jnp.float32) + b3_ref[...], 0.0)
        recon_ref[...] = jnp.dot(d, w4_ref[...],
                                 preferred_element_type=jnp.float32) + b4_ref[...]
        logit = jnp.dot(z, wc_ref[...],
                        preferred_element_type=jnp.float32) + bc_ref[...]
        cap_ref[...] = jax.nn.sigmoid(logit)

    return pl.pallas_call(
        body,
        grid=(_NB,),
        in_specs=[
            pl.BlockSpec((NC, _BLK, F), lambda i: (0, i, 0)),
            pl.BlockSpec((_BLK, F), lambda i: (i, 0)),
            pl.BlockSpec((_BLK, 1), lambda i: (i, 0)),
            pl.BlockSpec((1, HID), lambda i: (0, 0)),
            pl.BlockSpec((HID, LAT), lambda i: (0, 0)),
            pl.BlockSpec((1, LAT), lambda i: (0, 0)),
            pl.BlockSpec((HID, LAT), lambda i: (0, 0)),
            pl.BlockSpec((1, LAT), lambda i: (0, 0)),
            pl.BlockSpec((LAT, HID), lambda i: (0, 0)),
            pl.BlockSpec((1, HID), lambda i: (0, 0)),
            pl.BlockSpec((HID, VOCAB), lambda i: (0, 0)),
            pl.BlockSpec((1, VOCAB), lambda i: (0, 0)),
            pl.BlockSpec((LAT, 1), lambda i: (0, 0)),
            pl.BlockSpec((1, 1), lambda i: (0, 0)),
            pl.BlockSpec((_BLK, LAT), lambda i: (i, 0)),
        ],
        out_specs=(
            pl.BlockSpec((_BLK, VOCAB), lambda i: (i, 0)),
            pl.BlockSpec((_BLK, 1), lambda i: (i, 0)),
            pl.BlockSpec((_BLK, LAT), lambda i: (i, 0)),
            pl.BlockSpec((_BLK, LAT), lambda i: (i, 0)),
        ),
        out_shape=(
            jax.ShapeDtypeStruct((N, VOCAB), jnp.float32),
            jax.ShapeDtypeStruct((N, 1), jnp.float32),
            jax.ShapeDtypeStruct((N, LAT), jnp.float32),
            jax.ShapeDtypeStruct((N, LAT), jnp.float32),
        ),
    )(acc, hwp, dinv, b2, Wmu, bmu, Wlv, blv, W3, b3, W4, b4, Wc, bc, eps)


def kernel(x, edge_index, emb, W1, b1, W2, b2, Wmu, bmu, Wlv, blv,
           W3, b3, W4, b4, Wc, bc):
    x = x.astype(jnp.int32)
    src = edge_index[0].astype(jnp.int32)
    dst = edge_index[1].astype(jnp.int32)
    npad = CPT * 128 - EPT
    spread = jnp.arange(npad, dtype=jnp.int32) % 128
    pads = jnp.broadcast_to(spread, (NW, npad))
    padd = jnp.broadcast_to(N + spread, (NW, npad))
    src2d = jnp.concatenate([src.reshape(NW, EPT), pads],
                            axis=1).reshape(NW * CPT * 128)
    dst2d = jnp.concatenate([dst.reshape(NW, EPT), padd],
                            axis=1).reshape(NW * CPT * 128)

    T1 = _prep1(emb, W1)
    hist, rows = _deg_gather(dst, x, T1)
    dinv, hwp1 = _prep2(hist, rows)
    acc1 = _edge_scatter(src2d, dst2d, hwp1)
    hwp2 = _combine(acc1, hwp1, dinv, b1.reshape(1, -1), W2)
    acc2 = _edge_scatter(src2d, dst2d, hwp2)
    eps = jax.random.normal(jax.random.key(42), (N, LAT), jnp.float32)
    recon, cap, mu, logvar = _head(
        acc2, hwp2, dinv, b2.reshape(1, -1), Wmu, bmu.reshape(1, -1),
        Wlv, blv.reshape(1, -1), W3, b3.reshape(1, -1), W4, b4.reshape(1, -1),
        Wc, bc.reshape(1, -1), eps)
    return recon, cap, mu, logvar
